# baseline (speedup 1.0000x reference)
import functools

import jax
import jax.numpy as jnp
from jax import lax
from jax.experimental import pallas as pl
from jax.experimental.pallas import tpu as pltpu
from jax.experimental.pallas import tpu_sc as plsc

D_MODEL = 1024
N_ROWS = 32768
NC, NS = 2, 16
NW = NC * NS
ROWS_PER_W = N_ROWS // NW
CHUNK = 32
N_CHUNKS = ROWS_PER_W // CHUNK


def _make_gather():
    mesh = plsc.VectorSubcoreMesh(
        core_axis_name="c", subcore_axis_name="s",
        num_cores=NC, num_subcores=NS)

    @functools.partial(
        pl.kernel,
        out_type=jax.ShapeDtypeStruct((N_ROWS, D_MODEL), jnp.float32),
        mesh=mesh,
        scratch_types=[
            pltpu.VMEM((N_CHUNKS, CHUNK), jnp.int32),
            pltpu.VMEM((CHUNK, D_MODEL), jnp.float32),
            pltpu.VMEM((CHUNK, D_MODEL), jnp.float32),
            pltpu.VMEM_SHARED((NS, 1, CHUNK, D_MODEL), jnp.float32),
            pltpu.SemaphoreType.DMA,
            pltpu.SemaphoreType.DMA,
            pltpu.SemaphoreType.DMA,
            pltpu.SemaphoreType.DMA,
        ],
    )
    def gather_kernel(idx_hbm, table_hbm, out_hbm, idx_v, buf0, buf1,
                      stage, gsem0, gsem1, wsem0, wsem1):
        bufs = (buf0, buf1)
        gsems = (gsem0, gsem1)
        wsems = (wsem0, wsem1)
        sid = lax.axis_index("s")
        wid = sid * NC + lax.axis_index("c")
        base = wid * ROWS_PER_W
        pltpu.sync_copy(idx_hbm.at[wid], idx_v)

        def start_gather(j, b):
            pltpu.make_async_copy(
                table_hbm.at[idx_v.at[j]], bufs[b], gsems[b]).start()

        def wait_gather(j, b):
            pltpu.make_async_copy(
                table_hbm.at[idx_v.at[j]], bufs[b], gsems[b]).wait()

        def start_hbm_write(j, b):
            pltpu.make_async_copy(
                stage.at[sid, 0],
                out_hbm.at[pl.ds(base + j * CHUNK, CHUNK)], wsems[b]).start()

        def wait_hbm_write(j, b):
            pltpu.make_async_copy(
                stage.at[sid, 0],
                out_hbm.at[pl.ds(base + j * CHUNK, CHUNK)], wsems[b]).wait()

        start_gather(0, 0)
        start_gather(1, 1)

        def body(t, _):
            for b in range(2):
                j = 2 * t + b
                wait_gather(j, b)

                @pl.when(j > 0)
                def _():
                    wait_hbm_write(j - 1, 1 - b)

                pltpu.sync_copy(bufs[b], stage.at[sid, 0])
                start_hbm_write(j, b)

                @pl.when(j + 2 < N_CHUNKS)
                def _():
                    start_gather(j + 2, b)

            return ()

        lax.fori_loop(0, N_CHUNKS // 2, body, (), unroll=False)

        wait_hbm_write(N_CHUNKS - 1, (N_CHUNKS - 1) % 2)

    return gather_kernel


_gather = _make_gather()


def kernel(positions, pe):
    idx = positions.reshape(NW, N_CHUNKS, CHUNK).astype(jnp.int32)
    out = _gather(idx, pe)
    return out.reshape(positions.shape[0], positions.shape[1], D_MODEL)


# speedup vs baseline: 1.0118x; 1.0118x over previous
import functools

import jax
import jax.numpy as jnp
from jax import lax
from jax.experimental import pallas as pl
from jax.experimental.pallas import tpu as pltpu
from jax.experimental.pallas import tpu_sc as plsc

D_MODEL = 1024
N_ROWS = 32768
NC, NS = 2, 16
NW = NC * NS
ROWS_PER_W = N_ROWS // NW
CHUNK = 32
NBUF = 3
N_CHUNKS = ROWS_PER_W // CHUNK
N_FULL = (N_CHUNKS // NBUF) * NBUF
TAIL = N_CHUNKS - N_FULL


def _make_gather():
    mesh = plsc.VectorSubcoreMesh(
        core_axis_name="c", subcore_axis_name="s",
        num_cores=NC, num_subcores=NS)

    @functools.partial(
        pl.kernel,
        out_type=jax.ShapeDtypeStruct((N_ROWS, D_MODEL), jnp.float32),
        mesh=mesh,
        scratch_types=(
            [pltpu.VMEM((N_CHUNKS, CHUNK), jnp.int32)]
            + [pltpu.VMEM((CHUNK, D_MODEL), jnp.float32)] * NBUF
            + [pltpu.SemaphoreType.DMA] * NBUF
        ),
    )
    def gather_kernel(idx_hbm, table_hbm, out_hbm, idx_v, *bufs_and_sems):
        bufs = bufs_and_sems[:NBUF]
        gsems = bufs_and_sems[NBUF:]
        wid = lax.axis_index("s") * NC + lax.axis_index("c")
        base = wid * ROWS_PER_W
        pltpu.sync_copy(idx_hbm.at[wid], idx_v)

        def start_gather(j, b):
            pltpu.make_async_copy(
                table_hbm.at[idx_v.at[j]], bufs[b], gsems[b]).start()

        def wait_gather(j, b):
            pltpu.make_async_copy(
                table_hbm.at[idx_v.at[j]], bufs[b], gsems[b]).wait()

        def write_out(j, b):
            pltpu.sync_copy(bufs[b], out_hbm.at[pl.ds(base + j * CHUNK, CHUNK)])

        for b in range(NBUF):
            start_gather(b, b)

        def body(t, _):
            for b in range(NBUF):
                j = NBUF * t + b
                wait_gather(j, b)
                write_out(j, b)

                @pl.when(j + NBUF < N_CHUNKS)
                def _():
                    start_gather(j + NBUF, b)

            return ()

        lax.fori_loop(0, N_FULL // NBUF, body, (), unroll=False)

        for b in range(TAIL):
            j = N_FULL + b
            wait_gather(j, b)
            write_out(j, b)

    return gather_kernel


_gather = _make_gather()


def kernel(positions, pe):
    idx = positions.reshape(NW, N_CHUNKS, CHUNK).astype(jnp.int32)
    out = _gather(idx, pe)
    return out.reshape(positions.shape[0], positions.shape[1], D_MODEL)
